# baseline (speedup 1.0000x reference)
import functools

import jax
import jax.numpy as jnp
from jax import lax
from jax.experimental import pallas as pl
from jax.experimental.pallas import tpu as pltpu
from jax.experimental.pallas import tpu_sc as plsc

B = 16384
D = 8
NR = 62500
NC = 2
NS = 16
NW = NC * NS
BPW = B // NW
CHUNK = 128
NCHUNK = BPW // CHUNK


def _sc_gather_body(uidx_hbm, iidx_hbm, p_umlp, p_imlp, p_umf, p_imf,
                    o_umlp, o_imlp, o_umf, o_imf,
                    urow_v, ulane_v, irow_v, ilane_v,
                    gbuf, stage, sem):
    wid = lax.axis_index("s") * NC + lax.axis_index("c")
    base = wid * BPW
    lanes = lax.iota(jnp.int32, 16)
    half = lanes >> 3
    cvec = (lanes & 7) * 16

    for j in range(NCHUNK):
        pltpu.sync_copy(uidx_hbm.at[pl.ds(base + j * CHUNK, CHUNK)], urow_v.at[j])
        pltpu.sync_copy(iidx_hbm.at[pl.ds(base + j * CHUNK, CHUNK)], irow_v.at[j])

    def split(q, _):
        j = q // (CHUNK // 16)
        sl = pl.ds((q % (CHUNK // 16)) * 16, 16)
        uv = urow_v.at[j][sl]
        iv = irow_v.at[j][sl]
        ulane_v.at[j][sl] = uv & 15
        ilane_v.at[j][sl] = iv & 15
        urow_v.at[j][sl] = uv >> 4
        irow_v.at[j][sl] = iv >> 4
        return 0

    lax.fori_loop(0, BPW // 16, split, 0)

    tables = ((p_umlp, urow_v, ulane_v, o_umlp),
              (p_imlp, irow_v, ilane_v, o_imlp),
              (p_umf, urow_v, ulane_v, o_umf),
              (p_imf, irow_v, ilane_v, o_imf))
    for p_hbm, row_v, lane_v, o_hbm in tables:
        def chunk_body(j, _):
            pltpu.async_copy(p_hbm.at[row_v.at[j]], gbuf, sem).wait()
            for p in range(CHUNK // 2):
                loc = 2 * p + half
                m = plsc.load_gather(lane_v, [jnp.full((16,), j, jnp.int32), loc])
                vals = plsc.load_gather(gbuf, [loc, m + cvec])
                plsc.store_scatter(stage, [loc, lanes & 7], vals)
            pltpu.sync_copy(stage, o_hbm.at[pl.ds(base + j * CHUNK, CHUNK)])
            return 0

        lax.fori_loop(0, NCHUNK, chunk_body, 0)


_sc_gather = functools.partial(
    pl.kernel,
    out_type=[jax.ShapeDtypeStruct((B, 128), jnp.float32)] * 4,
    mesh=plsc.VectorSubcoreMesh(core_axis_name="c", subcore_axis_name="s"),
    compiler_params=pltpu.CompilerParams(
        use_tc_tiling_on_sc=False, needs_layout_passes=False),
    scratch_types=[
        pltpu.VMEM((NCHUNK, CHUNK), jnp.int32),
        pltpu.VMEM((NCHUNK, CHUNK), jnp.int32),
        pltpu.VMEM((NCHUNK, CHUNK), jnp.int32),
        pltpu.VMEM((NCHUNK, CHUNK), jnp.int32),
        pltpu.VMEM((CHUNK, 128), jnp.float32),
        pltpu.VMEM((CHUNK, 128), jnp.float32),
        pltpu.SemaphoreType.DMA,
    ],
)(_sc_gather_body)


BLK = 2048


def _tc_mlp_body(u_mlp, i_mlp, u_mf, i_mf,
                 w0u, w0i, b0, w1t, b1, w2t, b2, wa_mlp, wa_mf, ba,
                 out):
    xu = u_mlp[...][:, :D]
    xi = i_mlp[...][:, :D]
    h = xu @ w0u[...] + xi @ w0i[...] + b0[...]
    h = jnp.maximum(h, 0.0)
    h = jnp.maximum(h @ w1t[...] + b1[...], 0.0)
    h = jnp.maximum(h @ w2t[...] + b2[...], 0.0)
    mf = u_mf[...][:, :D] * i_mf[...][:, :D]
    out[...] = h @ wa_mlp[...] + mf @ wa_mf[...] + ba[...]


def _full(shape):
    return pl.BlockSpec(shape, lambda i: (0,) * len(shape))


def _pack(t):
    return t.T.reshape(D, NR, 16).transpose(1, 0, 2).reshape(NR, 128)


def kernel(user_indices, item_indices, emb_user_mlp, emb_item_mlp,
           emb_user_mf, emb_item_mf, W0, b0, W1, b1, W2, b2, Wa, ba):
    g_umlp, g_imlp, g_umf, g_imf = _sc_gather(
        user_indices, item_indices, _pack(emb_user_mlp), _pack(emb_item_mlp),
        _pack(emb_user_mf), _pack(emb_item_mf))

    w0u = W0[:, :D].T
    w0i = W0[:, D:].T
    w1t = W1.T
    w2t = W2.T
    wa_mlp = Wa[:, :8].T
    wa_mf = Wa[:, 8:].T
    b0r = b0.reshape(1, -1)
    b1r = b1.reshape(1, -1)
    b2r = b2.reshape(1, -1)
    bar = ba.reshape(1, -1)

    out = pl.pallas_call(
        _tc_mlp_body,
        grid=(B // BLK,),
        in_specs=[
            pl.BlockSpec((BLK, 128), lambda i: (i, 0)),
            pl.BlockSpec((BLK, 128), lambda i: (i, 0)),
            pl.BlockSpec((BLK, 128), lambda i: (i, 0)),
            pl.BlockSpec((BLK, 128), lambda i: (i, 0)),
            _full((D, 32)), _full((D, 32)), _full((1, 32)),
            _full((32, 16)), _full((1, 16)),
            _full((16, 8)), _full((1, 8)),
            _full((8, 1)), _full((8, 1)), _full((1, 1)),
        ],
        out_specs=pl.BlockSpec((BLK, 1), lambda i: (i, 0)),
        out_shape=jax.ShapeDtypeStruct((B, 1), jnp.float32),
    )(g_umlp, g_imlp, g_umf, g_imf,
      w0u, w0i, b0r, w1t, b1r, w2t, b2r, wa_mlp, wa_mf, bar)
    return out


# speedup vs baseline: 3.6987x; 3.6987x over previous
import functools

import jax
import jax.numpy as jnp
from jax import lax
from jax.experimental import pallas as pl
from jax.experimental.pallas import tpu as pltpu
from jax.experimental.pallas import tpu_sc as plsc

B = 16384
D = 8
N = 1000000
CB = 2048
NBLK = (N + CB - 1) // CB
NR = NBLK * 128
NC = 2
NS = 16
NW = NC * NS
BPW = B // NW
CHUNK = 128
NCHUNK = BPW // CHUNK


def _repack_body(t_ref, out_ref):
    x = t_ref[...]
    x128 = jnp.concatenate(
        [x[:, g * 128:(g + 1) * 128] for g in range(16)], axis=0)
    out_ref[...] = x128.T


def _pack(t):
    return pl.pallas_call(
        _repack_body,
        grid=(NBLK,),
        in_specs=[pl.BlockSpec((D, CB), lambda i: (0, i))],
        out_specs=pl.BlockSpec((128, 128), lambda i: (i, 0)),
        out_shape=jax.ShapeDtypeStruct((NR, 128), jnp.float32),
    )(t.T)


def _sc_gather_body(uidx_hbm, iidx_hbm, p_umlp, p_imlp, p_umf, p_imf,
                    o_umlp, o_imlp, o_umf, o_imf,
                    urow_v, ulane_v, irow_v, ilane_v,
                    gbuf, stage, sem):
    wid = lax.axis_index("s") * NC + lax.axis_index("c")
    base = wid * BPW
    lanes = lax.iota(jnp.int32, 16)
    half = lanes >> 3
    fvec = lanes & 7

    for j in range(NCHUNK):
        pltpu.sync_copy(uidx_hbm.at[pl.ds(base + j * CHUNK, CHUNK)], urow_v.at[j])
        pltpu.sync_copy(iidx_hbm.at[pl.ds(base + j * CHUNK, CHUNK)], irow_v.at[j])

    def split(q, _):
        j = q // (CHUNK // 16)
        sl = pl.ds((q % (CHUNK // 16)) * 16, 16)
        uv = urow_v.at[j][sl]
        iv = irow_v.at[j][sl]
        ulane_v.at[j][sl] = ((uv >> 7) & 15) * 8
        ilane_v.at[j][sl] = ((iv >> 7) & 15) * 8
        urow_v.at[j][sl] = ((uv >> 11) << 7) | (uv & 127)
        irow_v.at[j][sl] = ((iv >> 11) << 7) | (iv & 127)
        return 0

    lax.fori_loop(0, BPW // 16, split, 0)

    tables = ((p_umlp, urow_v, ulane_v, o_umlp),
              (p_imlp, irow_v, ilane_v, o_imlp),
              (p_umf, urow_v, ulane_v, o_umf),
              (p_imf, irow_v, ilane_v, o_imf))
    for p_hbm, row_v, lane_v, o_hbm in tables:
        def chunk_body(j, _):
            pltpu.async_copy(p_hbm.at[row_v.at[j]], gbuf, sem).wait()
            for p in range(CHUNK // 2):
                loc = 2 * p + half
                lb = plsc.load_gather(lane_v, [jnp.full((16,), j, jnp.int32), loc])
                vals = plsc.load_gather(gbuf, [loc, lb + fvec])
                plsc.store_scatter(stage, [loc, fvec], vals)
            pltpu.sync_copy(stage, o_hbm.at[pl.ds(base + j * CHUNK, CHUNK)])
            return 0

        lax.fori_loop(0, NCHUNK, chunk_body, 0)


_sc_gather = functools.partial(
    pl.kernel,
    out_type=[jax.ShapeDtypeStruct((B, 128), jnp.float32)] * 4,
    mesh=plsc.VectorSubcoreMesh(core_axis_name="c", subcore_axis_name="s"),
    compiler_params=pltpu.CompilerParams(
        use_tc_tiling_on_sc=False, needs_layout_passes=False),
    scratch_types=[
        pltpu.VMEM((NCHUNK, CHUNK), jnp.int32),
        pltpu.VMEM((NCHUNK, CHUNK), jnp.int32),
        pltpu.VMEM((NCHUNK, CHUNK), jnp.int32),
        pltpu.VMEM((NCHUNK, CHUNK), jnp.int32),
        pltpu.VMEM((CHUNK, 128), jnp.float32),
        pltpu.VMEM((CHUNK, 128), jnp.float32),
        pltpu.SemaphoreType.DMA,
    ],
)(_sc_gather_body)


BLK = 2048


def _tc_mlp_body(u_mlp, i_mlp, u_mf, i_mf,
                 w0u, w0i, b0, w1t, b1, w2t, b2, wa_mlp, wa_mf, ba,
                 out):
    xu = u_mlp[...][:, :D]
    xi = i_mlp[...][:, :D]
    h = xu @ w0u[...] + xi @ w0i[...] + b0[...]
    h = jnp.maximum(h, 0.0)
    h = jnp.maximum(h @ w1t[...] + b1[...], 0.0)
    h = jnp.maximum(h @ w2t[...] + b2[...], 0.0)
    mf = u_mf[...][:, :D] * i_mf[...][:, :D]
    out[...] = h @ wa_mlp[...] + mf @ wa_mf[...] + ba[...]


def _full(shape):
    return pl.BlockSpec(shape, lambda i: (0,) * len(shape))


def kernel(user_indices, item_indices, emb_user_mlp, emb_item_mlp,
           emb_user_mf, emb_item_mf, W0, b0, W1, b1, W2, b2, Wa, ba):
    g_umlp, g_imlp, g_umf, g_imf = _sc_gather(
        user_indices, item_indices, _pack(emb_user_mlp), _pack(emb_item_mlp),
        _pack(emb_user_mf), _pack(emb_item_mf))

    w0u = W0[:, :D].T
    w0i = W0[:, D:].T
    w1t = W1.T
    w2t = W2.T
    wa_mlp = Wa[:, :8].T
    wa_mf = Wa[:, 8:].T
    b0r = b0.reshape(1, -1)
    b1r = b1.reshape(1, -1)
    b2r = b2.reshape(1, -1)
    bar = ba.reshape(1, -1)

    out = pl.pallas_call(
        _tc_mlp_body,
        grid=(B // BLK,),
        in_specs=[
            pl.BlockSpec((BLK, 128), lambda i: (i, 0)),
            pl.BlockSpec((BLK, 128), lambda i: (i, 0)),
            pl.BlockSpec((BLK, 128), lambda i: (i, 0)),
            pl.BlockSpec((BLK, 128), lambda i: (i, 0)),
            _full((D, 32)), _full((D, 32)), _full((1, 32)),
            _full((32, 16)), _full((1, 16)),
            _full((16, 8)), _full((1, 8)),
            _full((8, 1)), _full((8, 1)), _full((1, 1)),
        ],
        out_specs=pl.BlockSpec((BLK, 1), lambda i: (i, 0)),
        out_shape=jax.ShapeDtypeStruct((B, 1), jnp.float32),
    )(g_umlp, g_imlp, g_umf, g_imf,
      w0u, w0i, b0r, w1t, b1r, w2t, b2r, wa_mlp, wa_mf, bar)
    return out


# speedup vs baseline: 18.5393x; 5.0123x over previous
import functools

import jax
import jax.numpy as jnp
from jax import lax
from jax.experimental import pallas as pl
from jax.experimental.pallas import tpu as pltpu
from jax.experimental.pallas import tpu_sc as plsc

B = 16384
D = 8
N = 1000000
CB = 2048
NBLK4 = (N + 4 * CB - 1) // (4 * CB)
NR = NBLK4 * 512
NC = 2
NS = 16
NW = NC * NS
BPW = B // NW
CHUNK = 128
NCHUNK = BPW // CHUNK


def _repack_body(*refs):
    t_refs, out_refs = refs[:4], refs[4:]
    for t_ref, out_ref in zip(t_refs, out_refs):
        x = t_ref[...]
        for sub in range(4):
            xs = x[:, sub * CB:(sub + 1) * CB]
            x128 = jnp.concatenate(
                [xs[:, g * 128:(g + 1) * 128] for g in range(16)], axis=0)
            out_ref[pl.ds(sub * 128, 128), :] = x128.T


def _pack4(t0, t1, t2, t3):
    return pl.pallas_call(
        _repack_body,
        grid=(NBLK4,),
        in_specs=[pl.BlockSpec((D, 4 * CB), lambda i: (0, i))] * 4,
        out_specs=[pl.BlockSpec((512, 128), lambda i: (i, 0))] * 4,
        out_shape=[jax.ShapeDtypeStruct((NR, 128), jnp.float32)] * 4,
    )(t0.T, t1.T, t2.T, t3.T)


def _sc_gather_body(uidx_hbm, iidx_hbm, p_umlp, p_imlp, p_umf, p_imf,
                    o_umlp, o_imlp, o_umf, o_imf,
                    urow_v, ulane_v, irow_v, ilane_v,
                    gbuf, stage, sem):
    wid = lax.axis_index("s") * NC + lax.axis_index("c")
    base = wid * BPW
    lanes = lax.iota(jnp.int32, 16)
    half = lanes >> 3
    fvec = lanes & 7

    for j in range(NCHUNK):
        pltpu.sync_copy(uidx_hbm.at[pl.ds(base + j * CHUNK, CHUNK)], urow_v.at[j])
        pltpu.sync_copy(iidx_hbm.at[pl.ds(base + j * CHUNK, CHUNK)], irow_v.at[j])

    def split(q, _):
        j = q // (CHUNK // 16)
        sl = pl.ds((q % (CHUNK // 16)) * 16, 16)
        uv = urow_v.at[j][sl]
        iv = irow_v.at[j][sl]
        ulane_v.at[j][sl] = ((uv >> 7) & 15) * 8
        ilane_v.at[j][sl] = ((iv >> 7) & 15) * 8
        urow_v.at[j][sl] = ((uv >> 11) << 7) | (uv & 127)
        irow_v.at[j][sl] = ((iv >> 11) << 7) | (iv & 127)
        return 0

    lax.fori_loop(0, BPW // 16, split, 0)

    tables = ((p_umlp, urow_v, ulane_v, o_umlp),
              (p_imlp, irow_v, ilane_v, o_imlp),
              (p_umf, urow_v, ulane_v, o_umf),
              (p_imf, irow_v, ilane_v, o_imf))
    for p_hbm, row_v, lane_v, o_hbm in tables:
        def chunk_body(j, _):
            pltpu.async_copy(p_hbm.at[row_v.at[j]], gbuf, sem).wait()
            for p in range(CHUNK // 2):
                loc = 2 * p + half
                lb = plsc.load_gather(lane_v, [jnp.full((16,), j, jnp.int32), loc])
                vals = plsc.load_gather(gbuf, [loc, lb + fvec])
                plsc.store_scatter(stage, [loc, fvec], vals)
            pltpu.sync_copy(stage, o_hbm.at[pl.ds(base + j * CHUNK, CHUNK)])
            return 0

        lax.fori_loop(0, NCHUNK, chunk_body, 0)


_sc_gather = functools.partial(
    pl.kernel,
    out_type=[jax.ShapeDtypeStruct((B, 128), jnp.float32)] * 4,
    mesh=plsc.VectorSubcoreMesh(core_axis_name="c", subcore_axis_name="s"),
    compiler_params=pltpu.CompilerParams(
        use_tc_tiling_on_sc=False, needs_layout_passes=False),
    scratch_types=[
        pltpu.VMEM((NCHUNK, CHUNK), jnp.int32),
        pltpu.VMEM((NCHUNK, CHUNK), jnp.int32),
        pltpu.VMEM((NCHUNK, CHUNK), jnp.int32),
        pltpu.VMEM((NCHUNK, CHUNK), jnp.int32),
        pltpu.VMEM((CHUNK, 128), jnp.float32),
        pltpu.VMEM((CHUNK, 128), jnp.float32),
        pltpu.SemaphoreType.DMA,
    ],
)(_sc_gather_body)


BLK = 2048


def _tc_mlp_body(u_mlp, i_mlp, u_mf, i_mf,
                 w0u, w0i, b0, w1t, b1, w2t, b2, wa_mlp, wa_mf, ba,
                 out):
    xu = u_mlp[...][:, :D]
    xi = i_mlp[...][:, :D]
    h = xu @ w0u[...] + xi @ w0i[...] + b0[...]
    h = jnp.maximum(h, 0.0)
    h = jnp.maximum(h @ w1t[...] + b1[...], 0.0)
    h = jnp.maximum(h @ w2t[...] + b2[...], 0.0)
    mf = u_mf[...][:, :D] * i_mf[...][:, :D]
    out[...] = h @ wa_mlp[...] + mf @ wa_mf[...] + ba[...]


def _full(shape):
    return pl.BlockSpec(shape, lambda i: (0,) * len(shape))


def kernel(user_indices, item_indices, emb_user_mlp, emb_item_mlp,
           emb_user_mf, emb_item_mf, W0, b0, W1, b1, W2, b2, Wa, ba):
    p_umlp, p_imlp, p_umf, p_imf = _pack4(
        emb_user_mlp, emb_item_mlp, emb_user_mf, emb_item_mf)
    g_umlp, g_imlp, g_umf, g_imf = _sc_gather(
        user_indices, item_indices, p_umlp, p_imlp, p_umf, p_imf)

    w0u = W0[:, :D].T
    w0i = W0[:, D:].T
    w1t = W1.T
    w2t = W2.T
    wa_mlp = Wa[:, :8].T
    wa_mf = Wa[:, 8:].T
    b0r = b0.reshape(1, -1)
    b1r = b1.reshape(1, -1)
    b2r = b2.reshape(1, -1)
    bar = ba.reshape(1, -1)

    out = pl.pallas_call(
        _tc_mlp_body,
        grid=(B // BLK,),
        in_specs=[
            pl.BlockSpec((BLK, 128), lambda i: (i, 0)),
            pl.BlockSpec((BLK, 128), lambda i: (i, 0)),
            pl.BlockSpec((BLK, 128), lambda i: (i, 0)),
            pl.BlockSpec((BLK, 128), lambda i: (i, 0)),
            _full((D, 32)), _full((D, 32)), _full((1, 32)),
            _full((32, 16)), _full((1, 16)),
            _full((16, 8)), _full((1, 8)),
            _full((8, 1)), _full((8, 1)), _full((1, 1)),
        ],
        out_specs=pl.BlockSpec((BLK, 1), lambda i: (i, 0)),
        out_shape=jax.ShapeDtypeStruct((B, 1), jnp.float32),
    )(g_umlp, g_imlp, g_umf, g_imf,
      w0u, w0i, b0r, w1t, b1r, w2t, b2r, wa_mlp, wa_mf, bar)
    return out


# speedup vs baseline: 22.4468x; 1.2108x over previous
import functools

import jax
import jax.numpy as jnp
from jax import lax
from jax.experimental import pallas as pl
from jax.experimental.pallas import tpu as pltpu
from jax.experimental.pallas import tpu_sc as plsc

B = 16384
D = 8
N = 1000000
CB = 2048
NBLK4 = (N + 8 * CB - 1) // (8 * CB)
NR = NBLK4 * 1024
NC = 2
NS = 16
NW = NC * NS
BPW = B // NW
CHUNK = 128
NCHUNK = BPW // CHUNK


def _repack_body(*refs):
    t_refs, out_refs = refs[:4], refs[4:]
    for t_ref, out_ref in zip(t_refs, out_refs):
        x = t_ref[...]
        for sub in range(8):
            xs = x[:, sub * CB:(sub + 1) * CB]
            x128 = jnp.concatenate(
                [xs[:, g * 128:(g + 1) * 128] for g in range(16)], axis=0)
            out_ref[pl.ds(sub * 128, 128), :] = x128.T


def _pack4(t0, t1, t2, t3):
    return pl.pallas_call(
        _repack_body,
        grid=(NBLK4,),
        in_specs=[pl.BlockSpec((D, 8 * CB), lambda i: (0, i))] * 4,
        out_specs=[pl.BlockSpec((1024, 128), lambda i: (i, 0))] * 4,
        out_shape=[jax.ShapeDtypeStruct((NR, 128), jnp.float32)] * 4,
    )(t0.T, t1.T, t2.T, t3.T)


def _sc_gather_body(uidx_hbm, iidx_hbm, p_umlp, p_imlp, p_umf, p_imf,
                    o_umlp, o_imlp, o_umf, o_imf,
                    urow_v, ulane_v, irow_v, ilane_v,
                    gbuf, stage, sem):
    wid = lax.axis_index("s") * NC + lax.axis_index("c")
    base = wid * BPW
    lanes = lax.iota(jnp.int32, 16)
    half = lanes >> 3
    fvec = lanes & 7

    for j in range(NCHUNK):
        pltpu.sync_copy(uidx_hbm.at[pl.ds(base + j * CHUNK, CHUNK)], urow_v.at[j])
        pltpu.sync_copy(iidx_hbm.at[pl.ds(base + j * CHUNK, CHUNK)], irow_v.at[j])

    def split(q, _):
        j = q // (CHUNK // 16)
        sl = pl.ds((q % (CHUNK // 16)) * 16, 16)
        uv = urow_v.at[j][sl]
        iv = irow_v.at[j][sl]
        ulane_v.at[j][sl] = ((uv >> 7) & 15) * 8
        ilane_v.at[j][sl] = ((iv >> 7) & 15) * 8
        urow_v.at[j][sl] = ((uv >> 11) << 7) | (uv & 127)
        irow_v.at[j][sl] = ((iv >> 11) << 7) | (iv & 127)
        return 0

    lax.fori_loop(0, BPW // 16, split, 0)

    tables = ((p_umlp, urow_v, ulane_v, o_umlp),
              (p_imlp, irow_v, ilane_v, o_imlp),
              (p_umf, urow_v, ulane_v, o_umf),
              (p_imf, irow_v, ilane_v, o_imf))
    for p_hbm, row_v, lane_v, o_hbm in tables:
        def chunk_body(j, _):
            pltpu.async_copy(p_hbm.at[row_v.at[j]], gbuf, sem).wait()
            for p in range(CHUNK // 2):
                loc = 2 * p + half
                lb = plsc.load_gather(lane_v, [jnp.full((16,), j, jnp.int32), loc])
                vals = plsc.load_gather(gbuf, [loc, lb + fvec])
                plsc.store_scatter(stage, [loc, fvec], vals)
            pltpu.sync_copy(stage, o_hbm.at[pl.ds(base + j * CHUNK, CHUNK)])
            return 0

        lax.fori_loop(0, NCHUNK, chunk_body, 0)


_sc_gather = functools.partial(
    pl.kernel,
    out_type=[jax.ShapeDtypeStruct((B, 128), jnp.float32)] * 4,
    mesh=plsc.VectorSubcoreMesh(core_axis_name="c", subcore_axis_name="s"),
    compiler_params=pltpu.CompilerParams(
        use_tc_tiling_on_sc=False, needs_layout_passes=False),
    scratch_types=[
        pltpu.VMEM((NCHUNK, CHUNK), jnp.int32),
        pltpu.VMEM((NCHUNK, CHUNK), jnp.int32),
        pltpu.VMEM((NCHUNK, CHUNK), jnp.int32),
        pltpu.VMEM((NCHUNK, CHUNK), jnp.int32),
        pltpu.VMEM((CHUNK, 128), jnp.float32),
        pltpu.VMEM((CHUNK, 128), jnp.float32),
        pltpu.SemaphoreType.DMA,
    ],
)(_sc_gather_body)


BLK = 2048


def _tc_mlp_body(u_mlp, i_mlp, u_mf, i_mf,
                 w0u, w0i, b0, w1t, b1, w2t, b2, wa_mlp, wa_mf, ba,
                 out):
    xu = u_mlp[...][:, :D]
    xi = i_mlp[...][:, :D]
    h = xu @ w0u[...] + xi @ w0i[...] + b0[...]
    h = jnp.maximum(h, 0.0)
    h = jnp.maximum(h @ w1t[...] + b1[...], 0.0)
    h = jnp.maximum(h @ w2t[...] + b2[...], 0.0)
    mf = u_mf[...][:, :D] * i_mf[...][:, :D]
    out[...] = h @ wa_mlp[...] + mf @ wa_mf[...] + ba[...]


def _full(shape):
    return pl.BlockSpec(shape, lambda i: (0,) * len(shape))


def kernel(user_indices, item_indices, emb_user_mlp, emb_item_mlp,
           emb_user_mf, emb_item_mf, W0, b0, W1, b1, W2, b2, Wa, ba):
    p_umlp, p_imlp, p_umf, p_imf = _pack4(
        emb_user_mlp, emb_item_mlp, emb_user_mf, emb_item_mf)
    g_umlp, g_imlp, g_umf, g_imf = _sc_gather(
        user_indices, item_indices, p_umlp, p_imlp, p_umf, p_imf)

    w0u = W0[:, :D].T
    w0i = W0[:, D:].T
    w1t = W1.T
    w2t = W2.T
    wa_mlp = Wa[:, :8].T
    wa_mf = Wa[:, 8:].T
    b0r = b0.reshape(1, -1)
    b1r = b1.reshape(1, -1)
    b2r = b2.reshape(1, -1)
    bar = ba.reshape(1, -1)

    out = pl.pallas_call(
        _tc_mlp_body,
        grid=(B // BLK,),
        in_specs=[
            pl.BlockSpec((BLK, 128), lambda i: (i, 0)),
            pl.BlockSpec((BLK, 128), lambda i: (i, 0)),
            pl.BlockSpec((BLK, 128), lambda i: (i, 0)),
            pl.BlockSpec((BLK, 128), lambda i: (i, 0)),
            _full((D, 32)), _full((D, 32)), _full((1, 32)),
            _full((32, 16)), _full((1, 16)),
            _full((16, 8)), _full((1, 8)),
            _full((8, 1)), _full((8, 1)), _full((1, 1)),
        ],
        out_specs=pl.BlockSpec((BLK, 1), lambda i: (i, 0)),
        out_shape=jax.ShapeDtypeStruct((B, 1), jnp.float32),
    )(g_umlp, g_imlp, g_umf, g_imf,
      w0u, w0i, b0r, w1t, b1r, w2t, b2r, wa_mlp, wa_mf, bar)
    return out


# speedup vs baseline: 24.2915x; 1.0822x over previous
import functools

import jax
import jax.numpy as jnp
from jax import lax
from jax.experimental import pallas as pl
from jax.experimental.pallas import tpu as pltpu
from jax.experimental.pallas import tpu_sc as plsc

B = 16384
D = 8
N = 1000000
CB = 2048
NBLK4 = (N + 16 * CB - 1) // (16 * CB)
NR = NBLK4 * 2048
NC = 2
NS = 16
NW = NC * NS
BPW = B // NW
CHUNK = 128
NCHUNK = BPW // CHUNK


def _repack_body(*refs):
    t_refs, out_refs = refs[:4], refs[4:]
    for t_ref, out_ref in zip(t_refs, out_refs):
        x = t_ref[...]
        for sub in range(16):
            xs = x[:, sub * CB:(sub + 1) * CB]
            x128 = jnp.concatenate(
                [xs[:, g * 128:(g + 1) * 128] for g in range(16)], axis=0)
            out_ref[pl.ds(sub * 128, 128), :] = x128.T


def _pack4(t0, t1, t2, t3):
    return pl.pallas_call(
        _repack_body,
        grid=(NBLK4,),
        in_specs=[pl.BlockSpec((D, 16 * CB), lambda i: (0, i))] * 4,
        out_specs=[pl.BlockSpec((2048, 128), lambda i: (i, 0))] * 4,
        out_shape=[jax.ShapeDtypeStruct((NR, 128), jnp.float32)] * 4,
    )(t0.T, t1.T, t2.T, t3.T)


def _sc_gather_body(uidx_hbm, iidx_hbm, p_umlp, p_imlp, p_umf, p_imf,
                    o_umlp, o_imlp, o_umf, o_imf,
                    urow_v, ulane_v, irow_v, ilane_v,
                    gbuf, stage, sem):
    wid = lax.axis_index("s") * NC + lax.axis_index("c")
    base = wid * BPW
    lanes = lax.iota(jnp.int32, 16)
    half = lanes >> 3
    fvec = lanes & 7

    for j in range(NCHUNK):
        pltpu.sync_copy(uidx_hbm.at[pl.ds(base + j * CHUNK, CHUNK)], urow_v.at[j])
        pltpu.sync_copy(iidx_hbm.at[pl.ds(base + j * CHUNK, CHUNK)], irow_v.at[j])

    def split(q, _):
        j = q // (CHUNK // 16)
        sl = pl.ds((q % (CHUNK // 16)) * 16, 16)
        uv = urow_v.at[j][sl]
        iv = irow_v.at[j][sl]
        ulane_v.at[j][sl] = ((uv >> 7) & 15) * 8
        ilane_v.at[j][sl] = ((iv >> 7) & 15) * 8
        urow_v.at[j][sl] = ((uv >> 11) << 7) | (uv & 127)
        irow_v.at[j][sl] = ((iv >> 11) << 7) | (iv & 127)
        return 0

    lax.fori_loop(0, BPW // 16, split, 0)

    tables = ((p_umlp, urow_v, ulane_v, o_umlp),
              (p_imlp, irow_v, ilane_v, o_imlp),
              (p_umf, urow_v, ulane_v, o_umf),
              (p_imf, irow_v, ilane_v, o_imf))
    for p_hbm, row_v, lane_v, o_hbm in tables:
        def chunk_body(j, _):
            pltpu.async_copy(p_hbm.at[row_v.at[j]], gbuf, sem).wait()
            for p in range(CHUNK // 2):
                loc = 2 * p + half
                lb = plsc.load_gather(lane_v, [jnp.full((16,), j, jnp.int32), loc])
                vals = plsc.load_gather(gbuf, [loc, lb + fvec])
                plsc.store_scatter(stage, [loc, fvec], vals)
            pltpu.sync_copy(stage, o_hbm.at[pl.ds(base + j * CHUNK, CHUNK)])
            return 0

        lax.fori_loop(0, NCHUNK, chunk_body, 0)


_sc_gather = functools.partial(
    pl.kernel,
    out_type=[jax.ShapeDtypeStruct((B, 128), jnp.float32)] * 4,
    mesh=plsc.VectorSubcoreMesh(core_axis_name="c", subcore_axis_name="s"),
    compiler_params=pltpu.CompilerParams(
        use_tc_tiling_on_sc=False, needs_layout_passes=False),
    scratch_types=[
        pltpu.VMEM((NCHUNK, CHUNK), jnp.int32),
        pltpu.VMEM((NCHUNK, CHUNK), jnp.int32),
        pltpu.VMEM((NCHUNK, CHUNK), jnp.int32),
        pltpu.VMEM((NCHUNK, CHUNK), jnp.int32),
        pltpu.VMEM((CHUNK, 128), jnp.float32),
        pltpu.VMEM((CHUNK, 128), jnp.float32),
        pltpu.SemaphoreType.DMA,
    ],
)(_sc_gather_body)


BLK = 2048


def _tc_mlp_body(u_mlp, i_mlp, u_mf, i_mf,
                 w0u, w0i, b0, w1t, b1, w2t, b2, wa_mlp, wa_mf, ba,
                 out):
    xu = u_mlp[...][:, :D]
    xi = i_mlp[...][:, :D]
    h = xu @ w0u[...] + xi @ w0i[...] + b0[...]
    h = jnp.maximum(h, 0.0)
    h = jnp.maximum(h @ w1t[...] + b1[...], 0.0)
    h = jnp.maximum(h @ w2t[...] + b2[...], 0.0)
    mf = u_mf[...][:, :D] * i_mf[...][:, :D]
    out[...] = h @ wa_mlp[...] + mf @ wa_mf[...] + ba[...]


def _full(shape):
    return pl.BlockSpec(shape, lambda i: (0,) * len(shape))


def kernel(user_indices, item_indices, emb_user_mlp, emb_item_mlp,
           emb_user_mf, emb_item_mf, W0, b0, W1, b1, W2, b2, Wa, ba):
    p_umlp, p_imlp, p_umf, p_imf = _pack4(
        emb_user_mlp, emb_item_mlp, emb_user_mf, emb_item_mf)
    g_umlp, g_imlp, g_umf, g_imf = _sc_gather(
        user_indices, item_indices, p_umlp, p_imlp, p_umf, p_imf)

    w0u = W0[:, :D].T
    w0i = W0[:, D:].T
    w1t = W1.T
    w2t = W2.T
    wa_mlp = Wa[:, :8].T
    wa_mf = Wa[:, 8:].T
    b0r = b0.reshape(1, -1)
    b1r = b1.reshape(1, -1)
    b2r = b2.reshape(1, -1)
    bar = ba.reshape(1, -1)

    out = pl.pallas_call(
        _tc_mlp_body,
        grid=(B // BLK,),
        in_specs=[
            pl.BlockSpec((BLK, 128), lambda i: (i, 0)),
            pl.BlockSpec((BLK, 128), lambda i: (i, 0)),
            pl.BlockSpec((BLK, 128), lambda i: (i, 0)),
            pl.BlockSpec((BLK, 128), lambda i: (i, 0)),
            _full((D, 32)), _full((D, 32)), _full((1, 32)),
            _full((32, 16)), _full((1, 16)),
            _full((16, 8)), _full((1, 8)),
            _full((8, 1)), _full((8, 1)), _full((1, 1)),
        ],
        out_specs=pl.BlockSpec((BLK, 1), lambda i: (i, 0)),
        out_shape=jax.ShapeDtypeStruct((B, 1), jnp.float32),
    )(g_umlp, g_imlp, g_umf, g_imf,
      w0u, w0i, b0r, w1t, b1r, w2t, b2r, wa_mlp, wa_mf, bar)
    return out
